# Initial kernel scaffold; baseline (speedup 1.0000x reference)
import jax
import jax.numpy as jnp
from jax.experimental import pallas as pl


def kernel(img, label):
    raise NotImplementedError("write your pallas kernel here")


# speedup vs baseline: 241.3025x; 241.3025x over previous
import functools

import jax
import jax.numpy as jnp
from jax import lax
from jax.experimental import pallas as pl
from jax.experimental.pallas import tpu as pltpu
from jax.experimental.pallas import tpu_sc as plsc

NC, NS, L = 2, 16, 16
NW = NC * NS
NCH = 96
CPW = NCH // NW
NPIX = 512 * 512
CHUNK = 16384
NCHUNK = NPIX // CHUNK
NVEC = CHUNK // L
NBINS = 256
NGRP = NBINS // L


def _eq_body(img_hbm, out_hbm, xb, ob, hist, excl, lut):
    c = lax.axis_index("c")
    s = lax.axis_index("s")
    wid = s * NC + c
    lanes = lax.iota(jnp.int32, L)
    ones = jnp.ones((L,), jnp.int32)

    def per_channel(ci, _):
        ch = wid * CPW + ci
        base = ch * NPIX

        def zero_row(b, _):
            hist[pl.ds(b * L, L)] = jnp.zeros((L,), jnp.int32)
            return 0
        lax.fori_loop(0, NBINS, zero_row, 0)

        def p1_chunk(k, _):
            pltpu.sync_copy(img_hbm.at[pl.ds(base + k * CHUNK, CHUNK)], xb)

            def p1_vec(i, _):
                x = xb[pl.ds(i * L, L)]
                q = jnp.clip((x * 256.0).astype(jnp.int32), 0, 255)
                plsc.addupdate_scatter(hist, [q * L + lanes], ones)
                return 0
            lax.fori_loop(0, NVEC, p1_vec, 0)
            return 0
        lax.fori_loop(0, NCHUNK, p1_chunk, 0)

        def lut_a(g, carry):
            run, cntv = carry
            rows = g * L + lanes
            t = jnp.zeros((L,), jnp.int32)

            def diag(l, t):
                cols = (lanes + l) & (L - 1)
                return t + plsc.load_gather(hist, [rows * L + cols])
            t = lax.fori_loop(0, L, diag, t)
            inc = plsc.cumsum(t)
            cdf_g = run + inc
            excl[pl.ds(g * L, L)] = cdf_g - t
            cntv = cntv + jnp.where(cdf_g == NPIX, 1, 0).astype(jnp.int32)
            return jnp.max(cdf_g), cntv

        run0 = jnp.int32(0)
        cnt0 = jnp.zeros((L,), jnp.int32)
        _, cntv = lax.fori_loop(0, NGRP, lut_a, (run0, cnt0))
        last = jnp.int32(NBINS) - jnp.sum(cntv)
        excl_last = plsc.load_gather(excl, [jnp.broadcast_to(last, (L,))])
        step = lax.div(excl_last, jnp.int32(255))
        safe = jnp.maximum(step, 1)
        half = lax.div(step, jnp.int32(2))

        def lut_b(g, _):
            e = excl[pl.ds(g * L, L)]
            lv = jnp.clip(lax.div(e + half, safe), 0, 255)
            bins = g * L + lanes
            li = jnp.where(step == 0, bins, lv)
            lut[pl.ds(g * L, L)] = li.astype(jnp.float32) * jnp.float32(1.0 / 255.0)
            return 0
        lax.fori_loop(0, NGRP, lut_b, 0)

        def p2_chunk(k, _):
            pltpu.sync_copy(img_hbm.at[pl.ds(base + k * CHUNK, CHUNK)], xb)

            def p2_vec(i, _):
                x = xb[pl.ds(i * L, L)]
                q = jnp.clip((x * 256.0).astype(jnp.int32), 0, 255)
                ob[pl.ds(i * L, L)] = plsc.load_gather(lut, [q])
                return 0
            lax.fori_loop(0, NVEC, p2_vec, 0)
            pltpu.sync_copy(ob, out_hbm.at[pl.ds(base + k * CHUNK, CHUNK)])
            return 0
        lax.fori_loop(0, NCHUNK, p2_chunk, 0)
        return 0

    lax.fori_loop(0, CPW, per_channel, 0)


_mesh = plsc.VectorSubcoreMesh(core_axis_name="c", subcore_axis_name="s")

_eq_call = functools.partial(
    pl.kernel,
    out_type=jax.ShapeDtypeStruct((NCH * NPIX,), jnp.float32),
    mesh=_mesh,
    compiler_params=pltpu.CompilerParams(needs_layout_passes=False),
    scratch_types=[
        pltpu.VMEM((CHUNK,), jnp.float32),
        pltpu.VMEM((CHUNK,), jnp.float32),
        pltpu.VMEM((NBINS * L,), jnp.int32),
        pltpu.VMEM((NBINS,), jnp.int32),
        pltpu.VMEM((NBINS,), jnp.float32),
    ],
)(_eq_body)


def kernel(img, label):
    B, C, H, W = img.shape
    out = _eq_call(img.reshape(-1))
    return out.reshape(B, C, H, W), label
